# baseline (speedup 1.0000x reference)
import functools

import jax
import jax.numpy as jnp
from jax import lax
from jax.experimental import pallas as pl
from jax.experimental.pallas import tpu as pltpu
from jax.experimental.pallas import tpu_sc as plsc

D = 128
G = 64
RB = 1024
NC = 2
NS = 16
CHUNK = 128

def _mesh():
    return plsc.VectorSubcoreMesh(core_axis_name="c", subcore_axis_name="s")


def _deg_body(per_tile, np_, dst_hbm, deg_out, idx_v, ones_v, zbuf_v, deg_sh):
    c = lax.axis_index("c")
    s = lax.axis_index("s")
    rpt = np_ // NS

    for k in range(CHUNK // 16):
        ones_v[pl.ds(k * 16, 16)] = jnp.ones((16,), jnp.float32)

    def zb(i, carry):
        zbuf_v[pl.ds(i * 16, 16)] = jnp.zeros((16,), jnp.float32)
        return carry
    lax.fori_loop(0, rpt // 16, zb, 0)
    pltpu.sync_copy(zbuf_v, deg_sh.at[pl.ds(s * rpt, rpt)])
    plsc.subcore_barrier()

    base = (c * NS + s) * per_tile

    def body(j, carry):
        off = base + j * CHUNK
        pltpu.sync_copy(dst_hbm.at[pl.ds(off, CHUNK)], idx_v)
        pltpu.sync_copy(ones_v, deg_sh.at[idx_v], add=True)
        return carry
    lax.fori_loop(0, per_tile // CHUNK, body, 0)

    plsc.subcore_barrier()
    pltpu.sync_copy(deg_sh.at[pl.ds(s * rpt, rpt)],
                    deg_out.at[c, pl.ds(s * rpt, rpt)])


def _make_deg(per_tile, np_):
    return pl.kernel(
        functools.partial(_deg_body, per_tile, np_),
        out_type=jax.ShapeDtypeStruct((NC, np_), jnp.float32),
        mesh=_mesh(),
        scratch_types=[
            pltpu.VMEM((CHUNK,), jnp.int32),
            pltpu.VMEM((CHUNK,), jnp.float32),
            pltpu.VMEM((np_ // NS,), jnp.float32),
            pltpu.VMEM_SHARED((np_,), jnp.float32),
        ],
    )


W = 16


def _mp_body(per_tile, np_, g_hbm, src2_hbm, dst2_hbm, out_hbm,
             idxs_v, idxd_v, bufs_v, acc_sh,
             sem_g0, sem_g1, sem_s0, sem_s1):
    sem_g = (sem_g0, sem_g1)
    sem_s = (sem_s0, sem_s1)
    c = lax.axis_index("c")
    s = lax.axis_index("s")
    rpt = np_ // NS
    nch = per_tile // CHUNK

    def wait_s(b):
        pltpu.make_async_copy(g_hbm.at[pl.ds(0, CHUNK)],
                              bufs_v.at[b], sem_s[b]).wait()

    def wait_g(b):
        pltpu.make_async_copy(g_hbm.at[pl.ds(0, CHUNK)],
                              bufs_v.at[b], sem_g[b]).wait()

    def zb(i, carry):
        for k in range(D // 16):
            bufs_v[0, i, pl.ds(k * 16, 16)] = jnp.zeros((16,), jnp.float32)
        return carry
    lax.fori_loop(0, CHUNK, zb, 0)

    def ib(j, carry):
        pltpu.sync_copy(bufs_v.at[0],
                        acc_sh.at[pl.ds(s * rpt + j * CHUNK, CHUNK)])
        return carry
    lax.fori_loop(0, rpt // CHUNK, ib, 0)
    plsc.subcore_barrier()

    row_base = (c * NS + s) * nch

    def body(sg, carry):
        pltpu.sync_copy(src2_hbm.at[pl.ds(row_base + sg * W, W)], idxs_v)
        pltpu.sync_copy(dst2_hbm.at[pl.ds(row_base + sg * W, W)], idxd_v)

        @pl.when(sg > 0)
        def _():
            wait_s(0)
        pltpu.async_copy(g_hbm.at[idxs_v.at[0]], bufs_v.at[0], sem_g[0])

        for w in range(W):
            b = w % 2
            bn = (w + 1) % 2
            if w < W - 1:
                if w == 0:
                    @pl.when(sg > 0)
                    def _():
                        wait_s(bn)
                else:
                    wait_s(bn)
                pltpu.async_copy(g_hbm.at[idxs_v.at[w + 1]],
                                 bufs_v.at[bn], sem_g[bn])
            wait_g(b)
            pltpu.async_copy(bufs_v.at[b], acc_sh.at[idxd_v.at[w]],
                             sem_s[b], add=True)
        return carry
    lax.fori_loop(0, nch // W, body, 0)

    wait_s(0)
    wait_s(1)

    plsc.subcore_barrier()
    pltpu.sync_copy(acc_sh.at[pl.ds(s * rpt, rpt)],
                    out_hbm.at[c, pl.ds(s * rpt, rpt)])


def _make_mp(per_tile, np_):
    return pl.kernel(
        functools.partial(_mp_body, per_tile, np_),
        out_type=jax.ShapeDtypeStruct((NC, np_, D), jnp.float32),
        mesh=_mesh(),
        scratch_types=[
            pltpu.VMEM((W, CHUNK), jnp.int32),
            pltpu.VMEM((W, CHUNK), jnp.int32),
            pltpu.VMEM((2, CHUNK, D), jnp.float32),
            pltpu.VMEM_SHARED((np_, D), jnp.float32),
        ] + [pltpu.SemaphoreType.DMA] * 4,
    )


def _dinv(deg_blk):
    d = deg_blk[:, 0:1] + deg_blk[:, 1:2] + 1.0
    return lax.rsqrt(d)


def _b1_body(x_ref, w_ref, deg_ref, g_ref):
    dinv = _dinv(deg_ref[...])
    g_ref[...] = dinv * jnp.dot(x_ref[...], w_ref[...],
                                preferred_element_type=jnp.float32)


def _bmid_body(m_ref, gp_ref, deg_ref, w_ref, b_ref, g_ref):
    dinv = _dinv(deg_ref[...])
    ssum = m_ref[0] + m_ref[1] + gp_ref[...]
    h = jnp.maximum(dinv * ssum + b_ref[...], 0.0)
    g_ref[...] = dinv * jnp.dot(h, w_ref[...],
                                preferred_element_type=jnp.float32)


def _pool_body(nrb, m_ref, g_ref, deg_ref, batch_ref, b3_ref, wl_ref, bl_ref,
               out_ref, sums, cnt):
    i = pl.program_id(0)

    @pl.when(i == 0)
    def _():
        sums[...] = jnp.zeros_like(sums)
        cnt[...] = jnp.zeros_like(cnt)

    dinv = _dinv(deg_ref[...])
    h3 = dinv * (m_ref[0] + m_ref[1] + g_ref[...]) + b3_ref[...]
    gid = lax.broadcasted_iota(jnp.int32, (G, RB), 0)
    mask = (batch_ref[0] == gid).astype(jnp.float32)
    sums[...] += jnp.dot(mask, h3, preferred_element_type=jnp.float32)
    cnt[...] += jnp.broadcast_to(jnp.sum(mask, axis=1, keepdims=True), (G, D))

    @pl.when(i == nrb - 1)
    def _():
        pooled = sums[...] / jnp.maximum(cnt[...], 1.0)
        out_ref[...] = jnp.dot(pooled, wl_ref[...],
                               preferred_element_type=jnp.float32) + bl_ref[...]


def kernel(x, edge_index, batch, W1, b1, W2, b2, W3, b3, Wl, bl):
    n = x.shape[0]
    e = edge_index.shape[1]
    out_dim = Wl.shape[1]
    np_ = -(-n // RB) * RB
    nrb = np_ // RB
    per_tile = -(-e // (NC * NS * W * CHUNK)) * W * CHUNK
    e_pad = per_tile * NC * NS

    src = jnp.concatenate(
        [edge_index[0], jnp.zeros((e_pad - e,), jnp.int32)])
    dst = jnp.concatenate(
        [edge_index[1], jnp.full((e_pad - e,), n, jnp.int32)])
    x_pad = jnp.pad(x, ((0, np_ - n), (0, 0)))
    batch3d = jnp.concatenate(
        [batch, jnp.full((np_ - n,), G, jnp.int32)]).reshape(nrb, 1, RB)
    b1r, b2r, b3r = (v.reshape(1, D) for v in (b1, b2, b3))
    wl_pad = jnp.pad(Wl, ((0, 0), (0, D - out_dim)))
    bl_pad = jnp.pad(bl, (0, D - out_dim)).reshape(1, D)

    deg = _make_deg(per_tile, np_)(dst)
    deg_cols = deg.T
    mp = _make_mp(per_tile, np_)

    row = lambda i: (i, 0)
    full = lambda i: (0, 0)
    spec_rd = pl.BlockSpec((RB, D), row)
    spec_m = pl.BlockSpec((NC, RB, D), lambda i: (0, i, 0))
    spec_deg = pl.BlockSpec((RB, 2), row)
    spec_w = pl.BlockSpec((D, D), full)
    spec_b = pl.BlockSpec((1, D), full)

    g1 = pl.pallas_call(
        _b1_body, grid=(nrb,),
        in_specs=[spec_rd, spec_w, spec_deg],
        out_specs=spec_rd,
        out_shape=jax.ShapeDtypeStruct((np_, D), jnp.float32),
    )(x_pad, W1, deg_cols)

    bmid = pl.pallas_call(
        _bmid_body, grid=(nrb,),
        in_specs=[spec_m, spec_rd, spec_deg, spec_w, spec_b],
        out_specs=spec_rd,
        out_shape=jax.ShapeDtypeStruct((np_, D), jnp.float32),
    )

    src2 = src.reshape(-1, CHUNK)
    dst2 = dst.reshape(-1, CHUNK)
    m1 = mp(g1, src2, dst2)
    g2 = bmid(m1, g1, deg_cols, W2, b1r)
    m2 = mp(g2, src2, dst2)
    g3 = bmid(m2, g2, deg_cols, W3, b2r)
    m3 = mp(g3, src2, dst2)

    out = pl.pallas_call(
        functools.partial(_pool_body, nrb), grid=(nrb,),
        in_specs=[spec_m, spec_rd, spec_deg,
                  pl.BlockSpec((1, 1, RB), lambda i: (i, 0, 0)),
                  spec_b, spec_w, spec_b],
        out_specs=pl.BlockSpec((G, D), full),
        out_shape=jax.ShapeDtypeStruct((G, D), jnp.float32),
        scratch_shapes=[pltpu.VMEM((G, D), jnp.float32),
                        pltpu.VMEM((G, D), jnp.float32)],
    )(m3, g3, deg_cols, batch3d, b3r, wl_pad, bl_pad)

    return out[:, :out_dim]


# speedup vs baseline: 3.1456x; 3.1456x over previous
import functools

import jax
import jax.numpy as jnp
from jax import lax
from jax.experimental import pallas as pl
from jax.experimental.pallas import tpu as pltpu
from jax.experimental.pallas import tpu_sc as plsc

D = 128
G = 64
RB = 1024
NC = 2
NS = 16
CHUNK = 128

def _mesh():
    return plsc.VectorSubcoreMesh(core_axis_name="c", subcore_axis_name="s")


def _deg_body(per_tile, np_, dst_hbm, deg_out, idx_v, ones_v, zbuf_v, deg_sh):
    c = lax.axis_index("c")
    s = lax.axis_index("s")
    rpt = np_ // NS

    for k in range(CHUNK // 16):
        ones_v[pl.ds(k * 16, 16)] = jnp.ones((16,), jnp.float32)

    def zb(i, carry):
        zbuf_v[pl.ds(i * 16, 16)] = jnp.zeros((16,), jnp.float32)
        return carry
    lax.fori_loop(0, rpt // 16, zb, 0)
    pltpu.sync_copy(zbuf_v, deg_sh.at[pl.ds(s * rpt, rpt)])
    plsc.subcore_barrier()

    base = (c * NS + s) * per_tile

    def body(j, carry):
        off = base + j * CHUNK
        pltpu.sync_copy(dst_hbm.at[pl.ds(off, CHUNK)], idx_v)
        pltpu.sync_copy(ones_v, deg_sh.at[idx_v], add=True)
        return carry
    lax.fori_loop(0, per_tile // CHUNK, body, 0)

    plsc.subcore_barrier()
    pltpu.sync_copy(deg_sh.at[pl.ds(s * rpt, rpt)],
                    deg_out.at[c, pl.ds(s * rpt, rpt)])


def _make_deg(per_tile, np_):
    return pl.kernel(
        functools.partial(_deg_body, per_tile, np_),
        out_type=jax.ShapeDtypeStruct((NC, np_), jnp.float32),
        mesh=_mesh(),
        scratch_types=[
            pltpu.VMEM((CHUNK,), jnp.int32),
            pltpu.VMEM((CHUNK,), jnp.float32),
            pltpu.VMEM((np_ // NS,), jnp.float32),
            pltpu.VMEM_SHARED((np_,), jnp.float32),
        ],
    )


W = 16


def _mp_body(per_tile, np_, g_hbm, src2_hbm, dst2_hbm, out_hbm,
             idxs_v, idxd_v, bufs_v, acc_sh,
             sem_g0, sem_g1, sem_s0, sem_s1):
    sem_g = (sem_g0, sem_g1)
    sem_s = (sem_s0, sem_s1)
    c = lax.axis_index("c")
    s = lax.axis_index("s")
    rpt = np_ // NS
    nch = per_tile // CHUNK

    def wait_s(b):
        pltpu.make_async_copy(g_hbm.at[pl.ds(0, CHUNK)],
                              bufs_v.at[b], sem_s[b]).wait()

    def wait_g(b):
        pltpu.make_async_copy(g_hbm.at[pl.ds(0, CHUNK)],
                              bufs_v.at[b], sem_g[b]).wait()

    def zb(i, carry):
        for k in range(D // 16):
            bufs_v[0, i, pl.ds(k * 16, 16)] = jnp.zeros((16,), jnp.float32)
        return carry
    lax.fori_loop(0, CHUNK, zb, 0)

    def ib(j, carry):
        pltpu.sync_copy(bufs_v.at[0],
                        acc_sh.at[pl.ds(s * rpt + j * CHUNK, CHUNK)])
        return carry
    lax.fori_loop(0, rpt // CHUNK, ib, 0)
    plsc.subcore_barrier()

    row_base = (c * NS + s) * nch

    def body(sg, carry):
        pltpu.sync_copy(src2_hbm.at[pl.ds(row_base + sg * W, W)], idxs_v)
        pltpu.sync_copy(dst2_hbm.at[pl.ds(row_base + sg * W, W)], idxd_v)

        @pl.when(sg > 0)
        def _():
            wait_s(0)
        pltpu.async_copy(g_hbm.at[idxs_v.at[0]], bufs_v.at[0], sem_g[0])

        for w in range(W):
            b = w % 2
            bn = (w + 1) % 2
            if w < W - 1:
                if w == 0:
                    @pl.when(sg > 0)
                    def _():
                        wait_s(bn)
                else:
                    wait_s(bn)
                pltpu.async_copy(g_hbm.at[idxs_v.at[w + 1]],
                                 bufs_v.at[bn], sem_g[bn])
            wait_g(b)
            pltpu.async_copy(bufs_v.at[b], acc_sh.at[idxd_v.at[w]],
                             sem_s[b], add=True)
        return carry
    lax.fori_loop(0, nch // W, body, 0)

    wait_s(0)
    wait_s(1)

    plsc.subcore_barrier()
    pltpu.sync_copy(acc_sh.at[pl.ds(s * rpt, rpt)],
                    out_hbm.at[c, pl.ds(s * rpt, rpt)])


def _make_mp(per_tile, np_):
    return pl.kernel(
        functools.partial(_mp_body, per_tile, np_),
        out_type=jax.ShapeDtypeStruct((NC, np_, D), jnp.float32),
        mesh=_mesh(),
        scratch_types=[
            pltpu.VMEM((W, CHUNK), jnp.int32),
            pltpu.VMEM((W, CHUNK), jnp.int32),
            pltpu.VMEM((2, CHUNK, D), jnp.float32),
            pltpu.VMEM_SHARED((np_, D), jnp.float32),
        ] + [pltpu.SemaphoreType.DMA] * 4,
    )


def _dinv(deg_blk):
    d = deg_blk[:, 0:1] + deg_blk[:, 1:2] + 1.0
    return lax.rsqrt(d)


def _b1_body(x_ref, w_ref, deg_ref, g_ref):
    dinv = _dinv(deg_ref[...])
    g_ref[...] = dinv * jnp.dot(x_ref[...], w_ref[...],
                                preferred_element_type=jnp.float32)


def _bmid_body(m_ref, gp_ref, deg_ref, w_ref, b_ref, g_ref):
    dinv = _dinv(deg_ref[...])
    ssum = m_ref[0] + m_ref[1] + gp_ref[...]
    h = jnp.maximum(dinv * ssum + b_ref[...], 0.0)
    g_ref[...] = dinv * jnp.dot(h, w_ref[...],
                                preferred_element_type=jnp.float32)


def _pool_body(nrb, m_ref, g_ref, deg_ref, batch_ref, b3_ref, wl_ref, bl_ref,
               out_ref, sums, cnt):
    i = pl.program_id(0)

    @pl.when(i == 0)
    def _():
        sums[...] = jnp.zeros_like(sums)
        cnt[...] = jnp.zeros_like(cnt)

    dinv = _dinv(deg_ref[...])
    h3 = dinv * (m_ref[0] + m_ref[1] + g_ref[...]) + b3_ref[...]
    gid = lax.broadcasted_iota(jnp.int32, (G, RB), 0)
    mask = (batch_ref[0] == gid).astype(jnp.float32)
    sums[...] += jnp.dot(mask, h3, preferred_element_type=jnp.float32)
    cnt[...] += jnp.broadcast_to(jnp.sum(mask, axis=1, keepdims=True), (G, D))

    @pl.when(i == nrb - 1)
    def _():
        pooled = sums[...] / jnp.maximum(cnt[...], 1.0)
        out_ref[...] = jnp.dot(pooled, wl_ref[...],
                               preferred_element_type=jnp.float32) + bl_ref[...]


def kernel(x, edge_index, batch, W1, b1, W2, b2, W3, b3, Wl, bl):
    n = x.shape[0]
    e = edge_index.shape[1]
    out_dim = Wl.shape[1]
    np_ = -(-n // RB) * RB
    nrb = np_ // RB
    per_tile = -(-e // (NC * NS * W * CHUNK)) * W * CHUNK
    e_pad = per_tile * NC * NS

    pad_ar = jnp.arange(e_pad - e, dtype=jnp.int32)
    src = jnp.concatenate([edge_index[0], pad_ar % n])
    dst = jnp.concatenate([edge_index[1], n + pad_ar % (np_ - n)])
    x_pad = jnp.pad(x, ((0, np_ - n), (0, 0)))
    batch3d = jnp.concatenate(
        [batch, jnp.full((np_ - n,), G, jnp.int32)]).reshape(nrb, 1, RB)
    b1r, b2r, b3r = (v.reshape(1, D) for v in (b1, b2, b3))
    wl_pad = jnp.pad(Wl, ((0, 0), (0, D - out_dim)))
    bl_pad = jnp.pad(bl, (0, D - out_dim)).reshape(1, D)

    deg = _make_deg(per_tile, np_)(dst)
    deg_cols = deg.T
    mp = _make_mp(per_tile, np_)

    row = lambda i: (i, 0)
    full = lambda i: (0, 0)
    spec_rd = pl.BlockSpec((RB, D), row)
    spec_m = pl.BlockSpec((NC, RB, D), lambda i: (0, i, 0))
    spec_deg = pl.BlockSpec((RB, 2), row)
    spec_w = pl.BlockSpec((D, D), full)
    spec_b = pl.BlockSpec((1, D), full)

    g1 = pl.pallas_call(
        _b1_body, grid=(nrb,),
        in_specs=[spec_rd, spec_w, spec_deg],
        out_specs=spec_rd,
        out_shape=jax.ShapeDtypeStruct((np_, D), jnp.float32),
    )(x_pad, W1, deg_cols)

    bmid = pl.pallas_call(
        _bmid_body, grid=(nrb,),
        in_specs=[spec_m, spec_rd, spec_deg, spec_w, spec_b],
        out_specs=spec_rd,
        out_shape=jax.ShapeDtypeStruct((np_, D), jnp.float32),
    )

    src2 = src.reshape(-1, CHUNK)
    dst2 = dst.reshape(-1, CHUNK)
    m1 = mp(g1, src2, dst2)
    g2 = bmid(m1, g1, deg_cols, W2, b1r)
    m2 = mp(g2, src2, dst2)
    g3 = bmid(m2, g2, deg_cols, W3, b2r)
    m3 = mp(g3, src2, dst2)

    out = pl.pallas_call(
        functools.partial(_pool_body, nrb), grid=(nrb,),
        in_specs=[spec_m, spec_rd, spec_deg,
                  pl.BlockSpec((1, 1, RB), lambda i: (i, 0, 0)),
                  spec_b, spec_w, spec_b],
        out_specs=pl.BlockSpec((G, D), full),
        out_shape=jax.ShapeDtypeStruct((G, D), jnp.float32),
        scratch_shapes=[pltpu.VMEM((G, D), jnp.float32),
                        pltpu.VMEM((G, D), jnp.float32)],
    )(m3, g3, deg_cols, batch3d, b3r, wl_pad, bl_pad)

    return out[:, :out_dim]


# speedup vs baseline: 3.3952x; 1.0793x over previous
import functools

import jax
import jax.numpy as jnp
from jax import lax
from jax.experimental import pallas as pl
from jax.experimental.pallas import tpu as pltpu
from jax.experimental.pallas import tpu_sc as plsc

D = 128
G = 64
RB = 1024
NC = 2
NS = 16
CHUNK = 128

def _mesh():
    return plsc.VectorSubcoreMesh(core_axis_name="c", subcore_axis_name="s")


DEG_GRP = 16


def _deg_body(per_tile, np_, dst2_hbm, deg_out, idx_v, ones_v, zbuf_v,
              deg_sh, sem):
    c = lax.axis_index("c")
    s = lax.axis_index("s")
    rpt = np_ // NS
    nch = per_tile // CHUNK
    ngr = nch // DEG_GRP

    for k in range(CHUNK // 16):
        ones_v[pl.ds(k * 16, 16)] = jnp.ones((16,), jnp.float32)

    def zb(i, carry):
        zbuf_v[pl.ds(i * 16, 16)] = jnp.zeros((16,), jnp.float32)
        return carry
    lax.fori_loop(0, rpt // 16, zb, 0)
    pltpu.sync_copy(zbuf_v, deg_sh.at[pl.ds(s * rpt, rpt)])

    row_base = (c * NS + s) * nch
    pltpu.sync_copy(dst2_hbm.at[pl.ds(row_base, nch)], idx_v)
    plsc.subcore_barrier()

    def drain(g):
        pltpu.make_async_copy(dst2_hbm.at[pl.ds(row_base, DEG_GRP)],
                              idx_v.at[pl.ds(0, DEG_GRP)], sem).wait()

    def fire(g):
        for w in range(DEG_GRP):
            pltpu.async_copy(ones_v, deg_sh.at[idx_v.at[g * DEG_GRP + w]],
                             sem, add=True)

    def body(g, carry):
        fire(g)
        drain(g - 1)
        return carry
    fire(0)
    lax.fori_loop(1, ngr, body, 0)
    drain(ngr - 1)

    plsc.subcore_barrier()
    pltpu.sync_copy(deg_sh.at[pl.ds(s * rpt, rpt)],
                    deg_out.at[c, pl.ds(s * rpt, rpt)])


def _make_deg(per_tile, np_):
    nch = per_tile // CHUNK
    return pl.kernel(
        functools.partial(_deg_body, per_tile, np_),
        out_type=jax.ShapeDtypeStruct((NC, np_), jnp.float32),
        mesh=_mesh(),
        scratch_types=[
            pltpu.VMEM((nch, CHUNK), jnp.int32),
            pltpu.VMEM((CHUNK,), jnp.float32),
            pltpu.VMEM((np_ // NS,), jnp.float32),
            pltpu.VMEM_SHARED((np_,), jnp.float32),
            pltpu.SemaphoreType.DMA,
        ],
    )


W = 16


def _mp_body(per_tile, np_, g_hbm, src2_hbm, dst2_hbm, out_hbm,
             idxs_v, idxd_v, bufs_v, acc_sh,
             sem_g0, sem_g1, sem_s0, sem_s1):
    sem_g = (sem_g0, sem_g1)
    sem_s = (sem_s0, sem_s1)
    c = lax.axis_index("c")
    s = lax.axis_index("s")
    rpt = np_ // NS
    nch = per_tile // CHUNK

    def wait_s(b):
        pltpu.make_async_copy(g_hbm.at[pl.ds(0, CHUNK)],
                              bufs_v.at[b], sem_s[b]).wait()

    def wait_g(b):
        pltpu.make_async_copy(g_hbm.at[pl.ds(0, CHUNK)],
                              bufs_v.at[b], sem_g[b]).wait()

    def zb(i, carry):
        for k in range(D // 16):
            bufs_v[0, i, pl.ds(k * 16, 16)] = jnp.zeros((16,), jnp.float32)
        return carry
    lax.fori_loop(0, CHUNK, zb, 0)

    def ib(j, carry):
        pltpu.sync_copy(bufs_v.at[0],
                        acc_sh.at[pl.ds(s * rpt + j * CHUNK, CHUNK)])
        return carry
    lax.fori_loop(0, rpt // CHUNK, ib, 0)
    plsc.subcore_barrier()

    row_base = (c * NS + s) * nch

    def body(sg, carry):
        pltpu.sync_copy(src2_hbm.at[pl.ds(row_base + sg * W, W)], idxs_v)
        pltpu.sync_copy(dst2_hbm.at[pl.ds(row_base + sg * W, W)], idxd_v)

        @pl.when(sg > 0)
        def _():
            wait_s(0)
        pltpu.async_copy(g_hbm.at[idxs_v.at[0]], bufs_v.at[0], sem_g[0])

        for w in range(W):
            b = w % 2
            bn = (w + 1) % 2
            if w < W - 1:
                if w == 0:
                    @pl.when(sg > 0)
                    def _():
                        wait_s(bn)
                else:
                    wait_s(bn)
                pltpu.async_copy(g_hbm.at[idxs_v.at[w + 1]],
                                 bufs_v.at[bn], sem_g[bn])
            wait_g(b)
            pltpu.async_copy(bufs_v.at[b], acc_sh.at[idxd_v.at[w]],
                             sem_s[b], add=True)
        return carry
    lax.fori_loop(0, nch // W, body, 0)

    wait_s(0)
    wait_s(1)

    plsc.subcore_barrier()
    pltpu.sync_copy(acc_sh.at[pl.ds(s * rpt, rpt)],
                    out_hbm.at[c, pl.ds(s * rpt, rpt)])


def _make_mp(per_tile, np_):
    return pl.kernel(
        functools.partial(_mp_body, per_tile, np_),
        out_type=jax.ShapeDtypeStruct((NC, np_, D), jnp.float32),
        mesh=_mesh(),
        scratch_types=[
            pltpu.VMEM((W, CHUNK), jnp.int32),
            pltpu.VMEM((W, CHUNK), jnp.int32),
            pltpu.VMEM((2, CHUNK, D), jnp.float32),
            pltpu.VMEM_SHARED((np_, D), jnp.float32),
        ] + [pltpu.SemaphoreType.DMA] * 4,
    )


def _dinv(deg_blk):
    d = deg_blk[:, 0:1] + deg_blk[:, 1:2] + 1.0
    return lax.rsqrt(d)


def _mm_body(x_ref, w_ref, t_ref):
    t_ref[...] = jnp.dot(x_ref[...], w_ref[...],
                         preferred_element_type=jnp.float32)


def _scale_body(t_ref, deg_ref, g_ref):
    g_ref[...] = _dinv(deg_ref[...]) * t_ref[...]


def _bmid_body(m_ref, gp_ref, deg_ref, w_ref, b_ref, g_ref):
    dinv = _dinv(deg_ref[...])
    ssum = m_ref[0] + m_ref[1] + gp_ref[...]
    h = jnp.maximum(dinv * ssum + b_ref[...], 0.0)
    g_ref[...] = dinv * jnp.dot(h, w_ref[...],
                                preferred_element_type=jnp.float32)


def _pool_body(nrb, m_ref, g_ref, deg_ref, batch_ref, b3_ref, wl_ref, bl_ref,
               out_ref, sums, cnt):
    i = pl.program_id(0)

    @pl.when(i == 0)
    def _():
        sums[...] = jnp.zeros_like(sums)
        cnt[...] = jnp.zeros_like(cnt)

    dinv = _dinv(deg_ref[...])
    h3 = dinv * (m_ref[0] + m_ref[1] + g_ref[...]) + b3_ref[...]
    gid = lax.broadcasted_iota(jnp.int32, (G, RB), 0)
    mask = (batch_ref[0] == gid).astype(jnp.float32)
    sums[...] += jnp.dot(mask, h3, preferred_element_type=jnp.float32)
    cnt[...] += jnp.broadcast_to(jnp.sum(mask, axis=1, keepdims=True), (G, D))

    @pl.when(i == nrb - 1)
    def _():
        pooled = sums[...] / jnp.maximum(cnt[...], 1.0)
        out_ref[...] = jnp.dot(pooled, wl_ref[...],
                               preferred_element_type=jnp.float32) + bl_ref[...]


def kernel(x, edge_index, batch, W1, b1, W2, b2, W3, b3, Wl, bl):
    n = x.shape[0]
    e = edge_index.shape[1]
    out_dim = Wl.shape[1]
    np_ = -(-n // RB) * RB
    nrb = np_ // RB
    per_tile = -(-e // (NC * NS * W * CHUNK)) * W * CHUNK
    e_pad = per_tile * NC * NS

    pad_ar = jnp.arange(e_pad - e, dtype=jnp.int32)
    src = jnp.concatenate([edge_index[0], pad_ar % n])
    dst = jnp.concatenate([edge_index[1], n + pad_ar % (np_ - n)])
    x_pad = jnp.pad(x, ((0, np_ - n), (0, 0)))
    batch3d = jnp.concatenate(
        [batch, jnp.full((np_ - n,), G, jnp.int32)]).reshape(nrb, 1, RB)
    b1r, b2r, b3r = (v.reshape(1, D) for v in (b1, b2, b3))
    wl_pad = jnp.pad(Wl, ((0, 0), (0, D - out_dim)))
    bl_pad = jnp.pad(bl, (0, D - out_dim)).reshape(1, D)

    src2 = src.reshape(-1, CHUNK)
    dst2 = dst.reshape(-1, CHUNK)
    deg = _make_deg(per_tile, np_)(dst2)
    deg_cols = deg.T
    mp = _make_mp(per_tile, np_)

    row = lambda i: (i, 0)
    full = lambda i: (0, 0)
    spec_rd = pl.BlockSpec((RB, D), row)
    spec_m = pl.BlockSpec((NC, RB, D), lambda i: (0, i, 0))
    spec_deg = pl.BlockSpec((RB, 2), row)
    spec_w = pl.BlockSpec((D, D), full)
    spec_b = pl.BlockSpec((1, D), full)

    t1 = pl.pallas_call(
        _mm_body, grid=(nrb,),
        in_specs=[spec_rd, spec_w],
        out_specs=spec_rd,
        out_shape=jax.ShapeDtypeStruct((np_, D), jnp.float32),
    )(x_pad, W1)
    g1 = pl.pallas_call(
        _scale_body, grid=(nrb,),
        in_specs=[spec_rd, spec_deg],
        out_specs=spec_rd,
        out_shape=jax.ShapeDtypeStruct((np_, D), jnp.float32),
    )(t1, deg_cols)

    bmid = pl.pallas_call(
        _bmid_body, grid=(nrb,),
        in_specs=[spec_m, spec_rd, spec_deg, spec_w, spec_b],
        out_specs=spec_rd,
        out_shape=jax.ShapeDtypeStruct((np_, D), jnp.float32),
    )

    m1 = mp(g1, src2, dst2)
    g2 = bmid(m1, g1, deg_cols, W2, b1r)
    m2 = mp(g2, src2, dst2)
    g3 = bmid(m2, g2, deg_cols, W3, b2r)
    m3 = mp(g3, src2, dst2)

    out = pl.pallas_call(
        functools.partial(_pool_body, nrb), grid=(nrb,),
        in_specs=[spec_m, spec_rd, spec_deg,
                  pl.BlockSpec((1, 1, RB), lambda i: (i, 0, 0)),
                  spec_b, spec_w, spec_b],
        out_specs=pl.BlockSpec((G, D), full),
        out_shape=jax.ShapeDtypeStruct((G, D), jnp.float32),
        scratch_shapes=[pltpu.VMEM((G, D), jnp.float32),
                        pltpu.VMEM((G, D), jnp.float32)],
    )(m3, g3, deg_cols, batch3d, b3r, wl_pad, bl_pad)

    return out[:, :out_dim]


# speedup vs baseline: 3.4984x; 1.0304x over previous
import functools

import jax
import jax.numpy as jnp
from jax import lax
from jax.experimental import pallas as pl
from jax.experimental.pallas import tpu as pltpu
from jax.experimental.pallas import tpu_sc as plsc

D = 128
G = 64
RB = 1024
NC = 2
NS = 16
CHUNK = 128

def _mesh():
    return plsc.VectorSubcoreMesh(core_axis_name="c", subcore_axis_name="s")


DEG_GRP = 16


def _deg_body(per_tile, np_, dst2_hbm, deg_out, idx_v, ones_v, zbuf_v,
              deg_sh, sem):
    c = lax.axis_index("c")
    s = lax.axis_index("s")
    rpt = np_ // NS
    nch = per_tile // CHUNK
    ngr = nch // DEG_GRP

    for k in range(CHUNK // 16):
        ones_v[pl.ds(k * 16, 16)] = jnp.ones((16,), jnp.float32)

    def zb(i, carry):
        zbuf_v[pl.ds(i * 16, 16)] = jnp.zeros((16,), jnp.float32)
        return carry
    lax.fori_loop(0, rpt // 16, zb, 0)
    pltpu.sync_copy(zbuf_v, deg_sh.at[pl.ds(s * rpt, rpt)])

    row_base = (c * NS + s) * nch
    pltpu.sync_copy(dst2_hbm.at[pl.ds(row_base, nch)], idx_v)
    plsc.subcore_barrier()

    def drain(g):
        pltpu.make_async_copy(dst2_hbm.at[pl.ds(row_base, DEG_GRP)],
                              idx_v.at[pl.ds(0, DEG_GRP)], sem).wait()

    def fire(g):
        for w in range(DEG_GRP):
            pltpu.async_copy(ones_v, deg_sh.at[idx_v.at[g * DEG_GRP + w]],
                             sem, add=True)

    def body(g, carry):
        fire(g)
        drain(g - 1)
        return carry
    fire(0)
    lax.fori_loop(1, ngr, body, 0)
    drain(ngr - 1)

    plsc.subcore_barrier()
    pltpu.sync_copy(deg_sh.at[pl.ds(s * rpt, rpt)],
                    deg_out.at[c, pl.ds(s * rpt, rpt)])


def _make_deg(per_tile, np_):
    nch = per_tile // CHUNK
    return pl.kernel(
        functools.partial(_deg_body, per_tile, np_),
        out_type=jax.ShapeDtypeStruct((NC, np_), jnp.float32),
        mesh=_mesh(),
        scratch_types=[
            pltpu.VMEM((nch, CHUNK), jnp.int32),
            pltpu.VMEM((CHUNK,), jnp.float32),
            pltpu.VMEM((np_ // NS,), jnp.float32),
            pltpu.VMEM_SHARED((np_,), jnp.float32),
            pltpu.SemaphoreType.DMA,
        ],
    )


W = 40


def _mp_body(per_tile, np_, g_hbm, src2_hbm, dst2_hbm, out_hbm,
             idxs_v, idxd_v, bufs_v, acc_sh,
             sem_g0, sem_g1, sem_s0, sem_s1):
    sem_g = (sem_g0, sem_g1)
    sem_s = (sem_s0, sem_s1)
    c = lax.axis_index("c")
    s = lax.axis_index("s")
    rpt = np_ // NS
    nch = per_tile // CHUNK

    def wait_s(b):
        pltpu.make_async_copy(g_hbm.at[pl.ds(0, CHUNK)],
                              bufs_v.at[b], sem_s[b]).wait()

    def wait_g(b):
        pltpu.make_async_copy(g_hbm.at[pl.ds(0, CHUNK)],
                              bufs_v.at[b], sem_g[b]).wait()

    @pl.when(c == 0)
    def _():
        pltpu.sync_copy(g_hbm.at[pl.ds(s * rpt, rpt)],
                        acc_sh.at[pl.ds(s * rpt, rpt)])

    @pl.when(c == 1)
    def _():
        def zb(i, carry):
            for k in range(D // 16):
                bufs_v[0, i, pl.ds(k * 16, 16)] = jnp.zeros((16,), jnp.float32)
            return carry
        lax.fori_loop(0, CHUNK, zb, 0)

        def ib(j, carry):
            pltpu.sync_copy(bufs_v.at[0],
                            acc_sh.at[pl.ds(s * rpt + j * CHUNK, CHUNK)])
            return carry
        lax.fori_loop(0, rpt // CHUNK, ib, 0)
    plsc.subcore_barrier()

    row_base = (c * NS + s) * nch

    def body(sg, carry):
        pltpu.sync_copy(src2_hbm.at[pl.ds(row_base + sg * W, W)], idxs_v)
        pltpu.sync_copy(dst2_hbm.at[pl.ds(row_base + sg * W, W)], idxd_v)

        @pl.when(sg > 0)
        def _():
            wait_s(0)
        pltpu.async_copy(g_hbm.at[idxs_v.at[0]], bufs_v.at[0], sem_g[0])

        for w in range(W):
            b = w % 2
            bn = (w + 1) % 2
            if w < W - 1:
                if w == 0:
                    @pl.when(sg > 0)
                    def _():
                        wait_s(bn)
                else:
                    wait_s(bn)
                pltpu.async_copy(g_hbm.at[idxs_v.at[w + 1]],
                                 bufs_v.at[bn], sem_g[bn])
            wait_g(b)
            pltpu.async_copy(bufs_v.at[b], acc_sh.at[idxd_v.at[w]],
                             sem_s[b], add=True)
        return carry
    lax.fori_loop(0, nch // W, body, 0)

    wait_s(0)
    wait_s(1)

    plsc.subcore_barrier()
    pltpu.sync_copy(acc_sh.at[pl.ds(s * rpt, rpt)],
                    out_hbm.at[c, pl.ds(s * rpt, rpt)])


def _make_mp(per_tile, np_):
    return pl.kernel(
        functools.partial(_mp_body, per_tile, np_),
        out_type=jax.ShapeDtypeStruct((NC, np_, D), jnp.float32),
        mesh=_mesh(),
        scratch_types=[
            pltpu.VMEM((W, CHUNK), jnp.int32),
            pltpu.VMEM((W, CHUNK), jnp.int32),
            pltpu.VMEM((2, CHUNK, D), jnp.float32),
            pltpu.VMEM_SHARED((np_, D), jnp.float32),
        ] + [pltpu.SemaphoreType.DMA] * 4,
    )


def _dinv(deg_blk):
    d = deg_blk[:, 0:1] + deg_blk[:, 1:2] + 1.0
    return lax.rsqrt(d)


def _mm_body(x_ref, w_ref, t_ref):
    t_ref[...] = jnp.dot(x_ref[...], w_ref[...],
                         preferred_element_type=jnp.float32)


def _scale_body(t_ref, deg_ref, g_ref):
    g_ref[...] = _dinv(deg_ref[...]) * t_ref[...]


def _bmid_body(m_ref, deg_ref, w_ref, b_ref, g_ref):
    dinv = _dinv(deg_ref[...])
    ssum = m_ref[0] + m_ref[1]
    h = jnp.maximum(dinv * ssum + b_ref[...], 0.0)
    g_ref[...] = dinv * jnp.dot(h, w_ref[...],
                                preferred_element_type=jnp.float32)


def _pool_body(nrb, m_ref, deg_ref, batch_ref, b3_ref, wl_ref, bl_ref,
               out_ref, sums, cnt):
    i = pl.program_id(0)

    @pl.when(i == 0)
    def _():
        sums[...] = jnp.zeros_like(sums)
        cnt[...] = jnp.zeros_like(cnt)

    dinv = _dinv(deg_ref[...])
    h3 = dinv * (m_ref[0] + m_ref[1]) + b3_ref[...]
    gid = lax.broadcasted_iota(jnp.int32, (G, RB), 0)
    mask = (batch_ref[0] == gid).astype(jnp.float32)
    sums[...] += jnp.dot(mask, h3, preferred_element_type=jnp.float32)
    cnt[...] += jnp.broadcast_to(jnp.sum(mask, axis=1, keepdims=True), (G, D))

    @pl.when(i == nrb - 1)
    def _():
        pooled = sums[...] / jnp.maximum(cnt[...], 1.0)
        out_ref[...] = jnp.dot(pooled, wl_ref[...],
                               preferred_element_type=jnp.float32) + bl_ref[...]


def kernel(x, edge_index, batch, W1, b1, W2, b2, W3, b3, Wl, bl):
    n = x.shape[0]
    e = edge_index.shape[1]
    out_dim = Wl.shape[1]
    np_ = -(-n // RB) * RB
    nrb = np_ // RB
    per_tile = -(-e // (NC * NS * W * CHUNK)) * W * CHUNK
    e_pad = per_tile * NC * NS

    pad_ar = jnp.arange(e_pad - e, dtype=jnp.int32)
    src = jnp.concatenate([edge_index[0], pad_ar % n])
    dst = jnp.concatenate([edge_index[1], n + pad_ar % (np_ - n)])
    x_pad = jnp.pad(x, ((0, np_ - n), (0, 0)))
    batch3d = jnp.concatenate(
        [batch, jnp.full((np_ - n,), G, jnp.int32)]).reshape(nrb, 1, RB)
    b1r, b2r, b3r = (v.reshape(1, D) for v in (b1, b2, b3))
    wl_pad = jnp.pad(Wl, ((0, 0), (0, D - out_dim)))
    bl_pad = jnp.pad(bl, (0, D - out_dim)).reshape(1, D)

    src2 = src.reshape(-1, CHUNK)
    dst2 = dst.reshape(-1, CHUNK)
    deg = _make_deg(per_tile, np_)(dst2)
    deg_cols = deg.T
    mp = _make_mp(per_tile, np_)

    row = lambda i: (i, 0)
    full = lambda i: (0, 0)
    spec_rd = pl.BlockSpec((RB, D), row)
    spec_m = pl.BlockSpec((NC, RB, D), lambda i: (0, i, 0))
    spec_deg = pl.BlockSpec((RB, 2), row)
    spec_w = pl.BlockSpec((D, D), full)
    spec_b = pl.BlockSpec((1, D), full)

    t1 = pl.pallas_call(
        _mm_body, grid=(nrb,),
        in_specs=[spec_rd, spec_w],
        out_specs=spec_rd,
        out_shape=jax.ShapeDtypeStruct((np_, D), jnp.float32),
    )(x_pad, W1)
    g1 = pl.pallas_call(
        _scale_body, grid=(nrb,),
        in_specs=[spec_rd, spec_deg],
        out_specs=spec_rd,
        out_shape=jax.ShapeDtypeStruct((np_, D), jnp.float32),
    )(t1, deg_cols)

    bmid = pl.pallas_call(
        _bmid_body, grid=(nrb,),
        in_specs=[spec_m, spec_deg, spec_w, spec_b],
        out_specs=spec_rd,
        out_shape=jax.ShapeDtypeStruct((np_, D), jnp.float32),
    )

    m1 = mp(g1, src2, dst2)
    g2 = bmid(m1, deg_cols, W2, b1r)
    m2 = mp(g2, src2, dst2)
    g3 = bmid(m2, deg_cols, W3, b2r)
    m3 = mp(g3, src2, dst2)

    out = pl.pallas_call(
        functools.partial(_pool_body, nrb), grid=(nrb,),
        in_specs=[spec_m, spec_deg,
                  pl.BlockSpec((1, 1, RB), lambda i: (i, 0, 0)),
                  spec_b, spec_w, spec_b],
        out_specs=pl.BlockSpec((G, D), full),
        out_shape=jax.ShapeDtypeStruct((G, D), jnp.float32),
        scratch_shapes=[pltpu.VMEM((G, D), jnp.float32),
                        pltpu.VMEM((G, D), jnp.float32)],
    )(m3, deg_cols, batch3d, b3r, wl_pad, bl_pad)

    return out[:, :out_dim]


# speedup vs baseline: 3.5783x; 1.0228x over previous
import functools

import jax
import jax.numpy as jnp
from jax import lax
from jax.experimental import pallas as pl
from jax.experimental.pallas import tpu as pltpu
from jax.experimental.pallas import tpu_sc as plsc

D = 128
G = 64
RB = 1024
NC = 2
NS = 16
CHUNK = 128

def _mesh():
    return plsc.VectorSubcoreMesh(core_axis_name="c", subcore_axis_name="s")


DEG_GRP = 16


def _deg_body(per_tile, np_, dst2_hbm, deg_out, idx_v, ones_v, zbuf_v,
              deg_sh, sem):
    c = lax.axis_index("c")
    s = lax.axis_index("s")
    rpt = np_ // NS
    nch = per_tile // CHUNK
    ngr = nch // DEG_GRP

    for k in range(CHUNK // 16):
        ones_v[pl.ds(k * 16, 16)] = jnp.ones((16,), jnp.float32)

    def zb(i, carry):
        zbuf_v[pl.ds(i * 16, 16)] = jnp.zeros((16,), jnp.float32)
        return carry
    lax.fori_loop(0, rpt // 16, zb, 0)
    pltpu.sync_copy(zbuf_v, deg_sh.at[pl.ds(s * rpt, rpt)])

    row_base = (c * NS + s) * nch
    pltpu.sync_copy(dst2_hbm.at[pl.ds(row_base, nch)], idx_v)
    plsc.subcore_barrier()

    def drain(g):
        pltpu.make_async_copy(dst2_hbm.at[pl.ds(row_base, DEG_GRP)],
                              idx_v.at[pl.ds(0, DEG_GRP)], sem).wait()

    def fire(g):
        for w in range(DEG_GRP):
            pltpu.async_copy(ones_v, deg_sh.at[idx_v.at[g * DEG_GRP + w]],
                             sem, add=True)

    def body(g, carry):
        fire(g)
        drain(g - 1)
        return carry
    fire(0)
    lax.fori_loop(1, ngr, body, 0)
    drain(ngr - 1)

    plsc.subcore_barrier()
    pltpu.sync_copy(deg_sh.at[pl.ds(s * rpt, rpt)],
                    deg_out.at[c, pl.ds(s * rpt, rpt)])


def _make_deg(per_tile, np_):
    nch = per_tile // CHUNK
    return pl.kernel(
        functools.partial(_deg_body, per_tile, np_),
        out_type=jax.ShapeDtypeStruct((NC, np_), jnp.float32),
        mesh=_mesh(),
        scratch_types=[
            pltpu.VMEM((nch, CHUNK), jnp.int32),
            pltpu.VMEM((CHUNK,), jnp.float32),
            pltpu.VMEM((np_ // NS,), jnp.float32),
            pltpu.VMEM_SHARED((np_,), jnp.float32),
            pltpu.SemaphoreType.DMA,
        ],
    )


W = 40


ZR = 16


def _mp_body(per_tile, np_, g_hbm, src2_hbm, dst2_hbm, out_hbm,
             idxs_v, idxd_v, bufs_v, zbuf_v, acc_sh,
             sem_g0, sem_g1, sem_s0, sem_s1, sem_i):
    sem_g = (sem_g0, sem_g1)
    sem_s = (sem_s0, sem_s1)
    c = lax.axis_index("c")
    s = lax.axis_index("s")
    rpt = np_ // NS
    nch = per_tile // CHUNK

    def wait_s(b):
        pltpu.make_async_copy(g_hbm.at[pl.ds(0, CHUNK)],
                              bufs_v.at[b], sem_s[b]).wait()

    def wait_g(b):
        pltpu.make_async_copy(g_hbm.at[pl.ds(0, CHUNK)],
                              bufs_v.at[b], sem_g[b]).wait()

    @pl.when(c == 0)
    def _():
        pltpu.async_copy(g_hbm.at[pl.ds(s * rpt, rpt)],
                         acc_sh.at[pl.ds(s * rpt, rpt)], sem_i)

    @pl.when(c == 1)
    def _():
        def zb(i, carry):
            for k in range(D // 16):
                zbuf_v[i, pl.ds(k * 16, 16)] = jnp.zeros((16,), jnp.float32)
            return carry
        lax.fori_loop(0, ZR, zb, 0)

        def ib(j, carry):
            pltpu.async_copy(zbuf_v,
                             acc_sh.at[pl.ds(s * rpt + j * ZR, ZR)], sem_i)
            return carry
        lax.fori_loop(0, rpt // ZR, ib, 0)

    row_base = (c * NS + s) * nch

    def body(sg, carry):
        pltpu.sync_copy(src2_hbm.at[pl.ds(row_base + sg * W, W)], idxs_v)
        pltpu.sync_copy(dst2_hbm.at[pl.ds(row_base + sg * W, W)], idxd_v)

        @pl.when(sg > 0)
        def _():
            wait_s(0)
        pltpu.async_copy(g_hbm.at[idxs_v.at[0]], bufs_v.at[0], sem_g[0])

        @pl.when(sg == 0)
        def _():
            pltpu.make_async_copy(g_hbm.at[pl.ds(0, rpt)],
                                  acc_sh.at[pl.ds(s * rpt, rpt)], sem_i).wait()
            plsc.subcore_barrier()

        for w in range(W):
            b = w % 2
            bn = (w + 1) % 2
            if w < W - 1:
                if w == 0:
                    @pl.when(sg > 0)
                    def _():
                        wait_s(bn)
                else:
                    wait_s(bn)
                pltpu.async_copy(g_hbm.at[idxs_v.at[w + 1]],
                                 bufs_v.at[bn], sem_g[bn])
            wait_g(b)
            pltpu.async_copy(bufs_v.at[b], acc_sh.at[idxd_v.at[w]],
                             sem_s[b], add=True)
        return carry
    lax.fori_loop(0, nch // W, body, 0)

    wait_s(0)
    wait_s(1)

    plsc.subcore_barrier()
    pltpu.sync_copy(acc_sh.at[pl.ds(s * rpt, rpt)],
                    out_hbm.at[c, pl.ds(s * rpt, rpt)])


def _make_mp(per_tile, np_):
    return pl.kernel(
        functools.partial(_mp_body, per_tile, np_),
        out_type=jax.ShapeDtypeStruct((NC, np_, D), jnp.float32),
        mesh=_mesh(),
        scratch_types=[
            pltpu.VMEM((W, CHUNK), jnp.int32),
            pltpu.VMEM((W, CHUNK), jnp.int32),
            pltpu.VMEM((2, CHUNK, D), jnp.float32),
            pltpu.VMEM((ZR, D), jnp.float32),
            pltpu.VMEM_SHARED((np_, D), jnp.float32),
        ] + [pltpu.SemaphoreType.DMA] * 5,
    )


def _dinv(deg_blk):
    d = deg_blk[:, 0:1] + deg_blk[:, 1:2] + 1.0
    return lax.rsqrt(d)


def _mm_body(x_ref, w_ref, t_ref):
    t_ref[...] = jnp.dot(x_ref[...], w_ref[...],
                         preferred_element_type=jnp.float32)


def _scale_body(t_ref, deg_ref, g_ref):
    g_ref[...] = _dinv(deg_ref[...]) * t_ref[...]


def _bmid_body(m_ref, deg_ref, w_ref, b_ref, g_ref):
    dinv = _dinv(deg_ref[...])
    ssum = m_ref[0] + m_ref[1]
    h = jnp.maximum(dinv * ssum + b_ref[...], 0.0)
    g_ref[...] = dinv * jnp.dot(h, w_ref[...],
                                preferred_element_type=jnp.float32)


def _pool_body(nrb, m_ref, deg_ref, batch_ref, b3_ref, wl_ref, bl_ref,
               out_ref, sums, cnt):
    i = pl.program_id(0)

    @pl.when(i == 0)
    def _():
        sums[...] = jnp.zeros_like(sums)
        cnt[...] = jnp.zeros_like(cnt)

    dinv = _dinv(deg_ref[...])
    h3 = dinv * (m_ref[0] + m_ref[1]) + b3_ref[...]
    gid = lax.broadcasted_iota(jnp.int32, (G, RB), 0)
    mask = (batch_ref[0] == gid).astype(jnp.float32)
    sums[...] += jnp.dot(mask, h3, preferred_element_type=jnp.float32)
    cnt[...] += jnp.broadcast_to(jnp.sum(mask, axis=1, keepdims=True), (G, D))

    @pl.when(i == nrb - 1)
    def _():
        pooled = sums[...] / jnp.maximum(cnt[...], 1.0)
        out_ref[...] = jnp.dot(pooled, wl_ref[...],
                               preferred_element_type=jnp.float32) + bl_ref[...]


def kernel(x, edge_index, batch, W1, b1, W2, b2, W3, b3, Wl, bl):
    n = x.shape[0]
    e = edge_index.shape[1]
    out_dim = Wl.shape[1]
    np_ = -(-n // RB) * RB
    nrb = np_ // RB
    per_tile = -(-e // (NC * NS * W * CHUNK)) * W * CHUNK
    e_pad = per_tile * NC * NS

    pad_ar = jnp.arange(e_pad - e, dtype=jnp.int32)
    src = jnp.concatenate([edge_index[0], pad_ar % n])
    dst = jnp.concatenate([edge_index[1], n + pad_ar % (np_ - n)])
    x_pad = jnp.pad(x, ((0, np_ - n), (0, 0)))
    batch3d = jnp.concatenate(
        [batch, jnp.full((np_ - n,), G, jnp.int32)]).reshape(nrb, 1, RB)
    b1r, b2r, b3r = (v.reshape(1, D) for v in (b1, b2, b3))
    wl_pad = jnp.pad(Wl, ((0, 0), (0, D - out_dim)))
    bl_pad = jnp.pad(bl, (0, D - out_dim)).reshape(1, D)

    src2 = src.reshape(-1, CHUNK)
    dst2 = dst.reshape(-1, CHUNK)
    deg = _make_deg(per_tile, np_)(dst2)
    deg_cols = deg.T
    mp = _make_mp(per_tile, np_)

    row = lambda i: (i, 0)
    full = lambda i: (0, 0)
    spec_rd = pl.BlockSpec((RB, D), row)
    spec_m = pl.BlockSpec((NC, RB, D), lambda i: (0, i, 0))
    spec_deg = pl.BlockSpec((RB, 2), row)
    spec_w = pl.BlockSpec((D, D), full)
    spec_b = pl.BlockSpec((1, D), full)

    t1 = pl.pallas_call(
        _mm_body, grid=(nrb,),
        in_specs=[spec_rd, spec_w],
        out_specs=spec_rd,
        out_shape=jax.ShapeDtypeStruct((np_, D), jnp.float32),
    )(x_pad, W1)
    g1 = pl.pallas_call(
        _scale_body, grid=(nrb,),
        in_specs=[spec_rd, spec_deg],
        out_specs=spec_rd,
        out_shape=jax.ShapeDtypeStruct((np_, D), jnp.float32),
    )(t1, deg_cols)

    bmid = pl.pallas_call(
        _bmid_body, grid=(nrb,),
        in_specs=[spec_m, spec_deg, spec_w, spec_b],
        out_specs=spec_rd,
        out_shape=jax.ShapeDtypeStruct((np_, D), jnp.float32),
    )

    m1 = mp(g1, src2, dst2)
    g2 = bmid(m1, deg_cols, W2, b1r)
    m2 = mp(g2, src2, dst2)
    g3 = bmid(m2, deg_cols, W3, b2r)
    m3 = mp(g3, src2, dst2)

    out = pl.pallas_call(
        functools.partial(_pool_body, nrb), grid=(nrb,),
        in_specs=[spec_m, spec_deg,
                  pl.BlockSpec((1, 1, RB), lambda i: (i, 0, 0)),
                  spec_b, spec_w, spec_b],
        out_specs=pl.BlockSpec((G, D), full),
        out_shape=jax.ShapeDtypeStruct((G, D), jnp.float32),
        scratch_shapes=[pltpu.VMEM((G, D), jnp.float32),
                        pltpu.VMEM((G, D), jnp.float32)],
    )(m3, deg_cols, batch3d, b3r, wl_pad, bl_pad)

    return out[:, :out_dim]
